# baseline (speedup 1.0000x reference)
import functools

import jax
import jax.numpy as jnp
from jax import lax
from jax.experimental import pallas as pl
from jax.experimental.pallas import tpu as pltpu
from jax.experimental.pallas import tpu_sc as plsc

_SUBSET = 8
_N = 2


def _num_reduced_states(m, n_max):
    import math

    return sum(math.comb(m + n - 1, n) for n in range(n_max + 1))


def kernel(rho, indices, num_segments):
    B, D, _ = rho.shape
    L = 16
    RB = D // 8
    HB = RB // 2
    NP = 13
    K = HB // NP
    VPP = K * 8 // L
    HE = HB * 8
    nseg = _num_reduced_states(_SUBSET, _N)
    seg_pad = -(-(nseg + 1) // 8) * 8

    rho4 = rho.reshape(B, RB, 8, D)
    segs = indices.astype(jnp.int32)

    mesh = plsc.VectorSubcoreMesh(core_axis_name="c", subcore_axis_name="s")

    @functools.partial(
        pl.kernel,
        mesh=mesh,
        out_type=jax.ShapeDtypeStruct((2, B, seg_pad), jnp.float32),
        scratch_types=[
            pltpu.VMEM((HE,), jnp.int32),
            pltpu.VMEM((2, K, 8, 128), jnp.float32),
            pltpu.VMEM((seg_pad * L,), jnp.float32),
            pltpu.VMEM((seg_pad,), jnp.float32),
            pltpu.SemaphoreType.DMA,
            pltpu.SemaphoreType.DMA,
        ],
        compiler_params=pltpu.CompilerParams(needs_layout_passes=False),
    )
    def run(rho_hbm, seg_hbm, out_hbm, seg_v, slab, acc, row_v, sem0, sem1):
        c = lax.axis_index("c")
        s = lax.axis_index("s")
        b = c * 8 + (s >> 1)
        h = s & 1
        base_e = h * HE
        base_R = h * HB
        sems = (sem0, sem1)

        def fire(p, slot):
            def one(t, carry):
                R = base_R + p * K + t
                w = pl.multiple_of((R >> 4) << 7, 128)
                pltpu.async_copy(
                    rho_hbm.at[b, R, :, pl.ds(w, 128)],
                    slab.at[slot, t],
                    sems[slot],
                )
                return carry

            lax.fori_loop(0, K, one, 0)

        def drain(slot):
            pltpu.make_async_copy(
                rho_hbm.at[0, pl.ds(0, K), :, pl.ds(0, 128)],
                slab.at[slot],
                sems[slot],
            ).wait()

        def zero_body(k, carry):
            acc[pl.ds(k * L, L)] = jnp.zeros((L,), jnp.float32)
            return carry

        lax.fori_loop(0, seg_pad, zero_body, 0)

        lane = lax.iota(jnp.int32, L)

        def consume(p, slot):
            slab_p = slab.at[slot]

            def body(kk, carry):
                j = base_e + (p * VPP + kk) * L + lane
                R = j >> 3
                d = j & 7
                rloc = R - (base_R + p * K)
                cloc = j & 127
                sg = seg_v[pl.ds((p * VPP + kk) * L, L)]
                v = plsc.load_gather(slab_p, [rloc, d, cloc])
                plsc.addupdate_scatter(acc, [sg * L + lane], jnp.abs(v))
                return carry

            lax.fori_loop(0, VPP, body, 0)

        fire(0, 0)
        pltpu.sync_copy(seg_hbm.at[pl.ds(base_e, HE)], seg_v)
        for p in range(NP):
            if p + 1 < NP:
                fire(p + 1, (p + 1) % 2)
            drain(p % 2)
            consume(p, p % 2)

        def fin(g, carry):
            def inner(jj, res):
                total = jnp.sum(acc[pl.ds((g * L + jj) * L, L)])
                return jnp.where(lane == jj, total, res)

            res = lax.fori_loop(0, L, inner, jnp.zeros((L,), jnp.float32))
            row_v[pl.ds(g * L, L)] = res
            return carry

        lax.fori_loop(0, seg_pad // L, fin, 0)

        pltpu.sync_copy(row_v, out_hbm.at[h, b])

    halves = run(rho4, segs)

    def combine(h_ref, o_ref):
        o_ref[...] = h_ref[0] + h_ref[1]

    out = pl.pallas_call(
        combine,
        out_shape=jax.ShapeDtypeStruct((B, seg_pad), jnp.float32),
    )(halves)
    return out[:, :nseg]


# speedup vs baseline: 1.1699x; 1.1699x over previous
import functools

import jax
import jax.numpy as jnp
from jax import lax
from jax.experimental import pallas as pl
from jax.experimental.pallas import tpu as pltpu
from jax.experimental.pallas import tpu_sc as plsc

_SUBSET = 8
_N = 2


def _num_reduced_states(m, n_max):
    import math

    return sum(math.comb(m + n - 1, n) for n in range(n_max + 1))


def kernel(rho, indices, num_segments):
    B, D, _ = rho.shape
    L = 16
    RB = D // 8
    HB = RB // 2
    NP = 5
    K = HB // NP
    VPP = K * 8 // L
    HE = HB * 8
    nseg = _num_reduced_states(_SUBSET, _N)
    seg_pad = -(-(nseg + 1) // 8) * 8

    rho4 = rho.reshape(B, RB, 8, D)
    segs = indices.astype(jnp.int32)

    mesh = plsc.VectorSubcoreMesh(core_axis_name="c", subcore_axis_name="s")

    @functools.partial(
        pl.kernel,
        mesh=mesh,
        out_type=jax.ShapeDtypeStruct((2, B, seg_pad), jnp.float32),
        scratch_types=[
            pltpu.VMEM((HE,), jnp.int32),
            pltpu.VMEM((3, K, 8, 128), jnp.float32),
            pltpu.VMEM((seg_pad * L,), jnp.float32),
            pltpu.VMEM((seg_pad,), jnp.float32),
            pltpu.SemaphoreType.DMA,
            pltpu.SemaphoreType.DMA,
            pltpu.SemaphoreType.DMA,
        ],
        compiler_params=pltpu.CompilerParams(needs_layout_passes=False),
    )
    def run(rho_hbm, seg_hbm, out_hbm, seg_v, slab, acc, row_v, sem0, sem1,
            sem2):
        c = lax.axis_index("c")
        s = lax.axis_index("s")
        b = c * 8 + (s >> 1)
        h = s & 1
        base_e = h * HE
        base_R = h * HB
        sems = (sem0, sem1, sem2)

        def fire(p, slot):
            def one(t, carry):
                R = base_R + p * K + t
                w = pl.multiple_of((R >> 4) << 7, 128)
                pltpu.async_copy(
                    rho_hbm.at[b, R, :, pl.ds(w, 128)],
                    slab.at[slot, t],
                    sems[slot],
                )
                return carry

            lax.fori_loop(0, K, one, 0)

        def drain(slot):
            pltpu.make_async_copy(
                rho_hbm.at[0, pl.ds(0, K), :, pl.ds(0, 128)],
                slab.at[slot],
                sems[slot],
            ).wait()

        def zero_body(k, carry):
            acc[pl.ds(k * L, L)] = jnp.zeros((L,), jnp.float32)
            return carry

        lax.fori_loop(0, seg_pad, zero_body, 0)

        lane = lax.iota(jnp.int32, L)

        def consume(p, slot):
            slab_p = slab.at[slot]

            def body(kk, carry):
                j = base_e + (p * VPP + kk) * L + lane
                R = j >> 3
                d = j & 7
                rloc = R - (base_R + p * K)
                cloc = j & 127
                sg = seg_v[pl.ds((p * VPP + kk) * L, L)]
                v = plsc.load_gather(slab_p, [rloc, d, cloc])
                plsc.addupdate_scatter(acc, [sg * L + lane], jnp.abs(v))
                return carry

            lax.fori_loop(0, VPP, body, 0)

        fire(0, 0)
        fire(1, 1)
        pltpu.sync_copy(seg_hbm.at[pl.ds(base_e, HE)], seg_v)
        for p in range(NP):
            if p + 2 < NP:
                fire(p + 2, (p + 2) % 3)
            drain(p % 3)
            consume(p, p % 3)

        def fin(g, carry):
            def inner(jj, res):
                total = jnp.sum(acc[pl.ds((g * L + jj) * L, L)])
                return jnp.where(lane == jj, total, res)

            res = lax.fori_loop(0, L, inner, jnp.zeros((L,), jnp.float32))
            row_v[pl.ds(g * L, L)] = res
            return carry

        lax.fori_loop(0, seg_pad // L, fin, 0)

        pltpu.sync_copy(row_v, out_hbm.at[h, b])

    halves = run(rho4, segs)

    def combine(h_ref, o_ref):
        o_ref[...] = h_ref[0] + h_ref[1]

    out = pl.pallas_call(
        combine,
        out_shape=jax.ShapeDtypeStruct((B, seg_pad), jnp.float32),
    )(halves)
    return out[:, :nseg]
